# baseline (speedup 1.0000x reference)
import functools

import jax
import jax.numpy as jnp
from jax import lax
from jax.experimental import pallas as pl
from jax.experimental.pallas import tpu as pltpu
from jax.experimental.pallas import tpu_sc as plsc

_DIM = 32
_WD = 0.01
_NC = 2
_NS = 16
_NW = _NC * _NS
_LANES = 16
_CHUNK = 128


def _sc_body(nchunks, u_hbm, i_hbm, j_hbm, W_hbm, H_hbm, x_hbm, reg_hbm,
             idx_u, idx_i, idx_j, u_r, i_r, j_r, x_v, reg_v, sem):
    wid = lax.axis_index("s") * _NC + lax.axis_index("c")
    bpw = nchunks * _CHUNK
    base = wid * bpw

    cps = []
    for k in range(nchunks):
        sl = pl.ds(base + k * _CHUNK, _CHUNK)
        cps.append(pltpu.async_copy(u_hbm.at[sl], idx_u.at[k], sem))
        cps.append(pltpu.async_copy(i_hbm.at[sl], idx_i.at[k], sem))
        cps.append(pltpu.async_copy(j_hbm.at[sl], idx_j.at[k], sem))
    for cp in cps:
        cp.wait()

    lane = lax.broadcasted_iota(jnp.int32, (_LANES,), 0)

    def pass_body(p, sq0):
        cps = [
            pltpu.async_copy(W_hbm.at[idx_u.at[p]], u_r, sem),
            pltpu.async_copy(H_hbm.at[idx_i.at[p]], i_r, sem),
            pltpu.async_copy(H_hbm.at[idx_j.at[p]], j_r, sem),
        ]
        for cp in cps:
            cp.wait()

        def group(gg, sq):
            x_v[pl.ds(p * _CHUNK + gg * _LANES, _LANES)] = sq
            return sq

        return lax.fori_loop(0, _CHUNK // _LANES, group, sq0)

    reg = lax.fori_loop(0, nchunks, pass_body,
                        jnp.zeros((_LANES,), jnp.float32))
    reg_v[...] = reg
    pltpu.sync_copy(x_v, x_hbm.at[pl.ds(base, bpw)])
    pltpu.sync_copy(reg_v, reg_hbm.at[wid])


def _tc_body(x_ref, reg_ref, out_ref):
    xs = x_ref[...]
    ls = jnp.minimum(xs, 0.0) - jnp.log1p(jnp.exp(-jnp.abs(xs)))
    out_ref[0, 0] = -jnp.sum(ls) + _WD * jnp.sum(reg_ref[...])


def kernel(u, i, j, W, H):
    B = u.shape[0]
    nchunks = B // (_NW * _CHUNK)
    bpw = nchunks * _CHUNK
    mesh = plsc.VectorSubcoreMesh(core_axis_name="c", subcore_axis_name="s")

    sc = pl.kernel(
        functools.partial(_sc_body, nchunks),
        out_type=(
            jax.ShapeDtypeStruct((B,), jnp.float32),
            jax.ShapeDtypeStruct((_NW, _LANES), jnp.float32),
        ),
        mesh=mesh,
        compiler_params=pltpu.CompilerParams(
            needs_layout_passes=False, use_tc_tiling_on_sc=False),
        scratch_types=[
            pltpu.VMEM((nchunks, _CHUNK), jnp.int32),
            pltpu.VMEM((nchunks, _CHUNK), jnp.int32),
            pltpu.VMEM((nchunks, _CHUNK), jnp.int32),
            pltpu.VMEM((_CHUNK, _DIM), jnp.bfloat16),
            pltpu.VMEM((_CHUNK, _DIM), jnp.bfloat16),
            pltpu.VMEM((_CHUNK, _DIM), jnp.bfloat16),
            pltpu.VMEM((bpw,), jnp.float32),
            pltpu.VMEM((_LANES,), jnp.float32),
            pltpu.SemaphoreType.DMA,
        ],
    )

    x, reg = sc(u.astype(jnp.int32), i.astype(jnp.int32), j.astype(jnp.int32),
                W.astype(jnp.bfloat16), H.astype(jnp.bfloat16))

    out = pl.pallas_call(
        _tc_body,
        out_shape=jax.ShapeDtypeStruct((1, 1), jnp.float32),
        out_specs=pl.BlockSpec(memory_space=pltpu.SMEM),
    )(x.reshape(B // 128, 128), reg)
    return out.reshape(())


# speedup vs baseline: 1.1387x; 1.1387x over previous
import functools

import jax
import jax.numpy as jnp
from jax import lax
from jax.experimental import pallas as pl
from jax.experimental.pallas import tpu as pltpu
from jax.experimental.pallas import tpu_sc as plsc

_DIM = 32
_WD = 0.01
_NC = 2
_NS = 16
_NW = _NC * _NS
_LANES = 16
_CHUNK = 128


def _sc_body(nchunks, u_hbm, i_hbm, j_hbm, W_hbm, H_hbm, x_hbm, reg_hbm,
             idx_u, idx_i, idx_j, u_r, i_r, j_r, x_v, reg_v, sem):
    wid = lax.axis_index("s") * _NC + lax.axis_index("c")
    bpw = nchunks * _CHUNK
    base = wid * bpw

    cps = []
    for k in range(nchunks):
        sl = pl.ds(base + k * _CHUNK, _CHUNK)
        cps.append(pltpu.async_copy(u_hbm.at[sl], idx_u.at[k], sem))
        cps.append(pltpu.async_copy(i_hbm.at[sl], idx_i.at[k], sem))
        cps.append(pltpu.async_copy(j_hbm.at[sl], idx_j.at[k], sem))
    for cp in cps:
        cp.wait()

    lane = lax.broadcasted_iota(jnp.int32, (_LANES,), 0)

    def pass_body(p, sq0):
        cps = [
            pltpu.async_copy(W_hbm.at[idx_u.at[p]], u_r, sem),
            pltpu.async_copy(H_hbm.at[idx_i.at[p]], i_r, sem),
            pltpu.async_copy(H_hbm.at[idx_j.at[p]], j_r, sem),
        ]
        for cp in cps:
            cp.wait()

        def group(gg, sq):
            rowi = lane + gg * _LANES
            acc_ui = jnp.zeros((_LANES,), jnp.float32)
            acc_uj = jnp.zeros((_LANES,), jnp.float32)
            for d in range(_DIM):
                col = jnp.full((_LANES,), d, jnp.int32)
                uc = plsc.load_gather(u_r, [rowi, col])
                ic = plsc.load_gather(i_r, [rowi, col])
                jc = plsc.load_gather(j_r, [rowi, col])
                acc_ui = acc_ui + uc * ic
                acc_uj = acc_uj + uc * jc
                sq = sq + (uc * uc + ic * ic + jc * jc)
            x_v[pl.ds(p * _CHUNK + gg * _LANES, _LANES)] = acc_ui - acc_uj
            return sq

        return lax.fori_loop(0, _CHUNK // _LANES, group, sq0)

    reg = lax.fori_loop(0, nchunks, pass_body,
                        jnp.zeros((_LANES,), jnp.float32))
    reg_v[...] = reg
    pltpu.sync_copy(x_v, x_hbm.at[pl.ds(base, bpw)])
    pltpu.sync_copy(reg_v, reg_hbm.at[wid])


def _tc_body(x_ref, reg_ref, out_ref):
    xs = x_ref[...]
    ls = jnp.minimum(xs, 0.0) - jnp.log1p(jnp.exp(-jnp.abs(xs)))
    out_ref[0, 0] = -jnp.sum(ls) + _WD * jnp.sum(reg_ref[...])


def kernel(u, i, j, W, H):
    B = u.shape[0]
    nchunks = B // (_NW * _CHUNK)
    bpw = nchunks * _CHUNK
    mesh = plsc.VectorSubcoreMesh(core_axis_name="c", subcore_axis_name="s")

    sc = pl.kernel(
        functools.partial(_sc_body, nchunks),
        out_type=(
            jax.ShapeDtypeStruct((B,), jnp.float32),
            jax.ShapeDtypeStruct((_NW, _LANES), jnp.float32),
        ),
        mesh=mesh,
        compiler_params=pltpu.CompilerParams(
            needs_layout_passes=False, use_tc_tiling_on_sc=False),
        scratch_types=[
            pltpu.VMEM((nchunks, _CHUNK), jnp.int32),
            pltpu.VMEM((nchunks, _CHUNK), jnp.int32),
            pltpu.VMEM((nchunks, _CHUNK), jnp.int32),
            pltpu.VMEM((_CHUNK, _DIM), jnp.float32),
            pltpu.VMEM((_CHUNK, _DIM), jnp.float32),
            pltpu.VMEM((_CHUNK, _DIM), jnp.float32),
            pltpu.VMEM((bpw,), jnp.float32),
            pltpu.VMEM((_LANES,), jnp.float32),
            pltpu.SemaphoreType.DMA,
        ],
    )

    x, reg = sc(u.astype(jnp.int32), i.astype(jnp.int32), j.astype(jnp.int32),
                W, H)

    out = pl.pallas_call(
        _tc_body,
        out_shape=jax.ShapeDtypeStruct((1, 1), jnp.float32),
        out_specs=pl.BlockSpec(memory_space=pltpu.SMEM),
    )(x.reshape(B // 128, 128), reg)
    return out.reshape(())
